# baseline (speedup 1.0000x reference)
import functools

import jax
import jax.numpy as jnp
from jax import lax
from jax.experimental import pallas as pl
from jax.experimental.pallas import tpu as pltpu
from jax.experimental.pallas import tpu_sc as plsc

NC = 2
NS = 16
CH = 80


def _build_sc_aggregate(NP, D, R, NCHK):
    ZPT = NP // NS
    DROW = NP // 128

    mesh = plsc.VectorSubcoreMesh(core_axis_name="c", subcore_axis_name="s")

    out_type = [
        jax.ShapeDtypeStruct((NC, NP, D), jnp.float32),
        jax.ShapeDtypeStruct((NC * NS, DROW, 128), jnp.float32),
    ]
    scratch = [
        pltpu.VMEM((2, 3, CH), jnp.int32),
        pltpu.VMEM((CH, D), jnp.float32),
        pltpu.VMEM((2, CH, D), jnp.float32),
        pltpu.VMEM((DROW, 128), jnp.float32),
        pltpu.VMEM_SHARED((NP, D), jnp.float32),
        pltpu.SemaphoreType.DMA,
        pltpu.SemaphoreType.DMA,
    ]

    @functools.partial(pl.kernel, out_type=out_type, mesh=mesh,
                       scratch_types=scratch)
    def sc_kernel(x_hbm, idx_hbm, rel_hbm, z_hbm, agg_out, deg_out,
                  idx_v, relbuf, xbuf, dloc, acc_sp, sem, sem2):
        cid = lax.axis_index("c")
        sid = lax.axis_index("s")
        wid = cid * NS + sid

        zv = jnp.zeros((16,), jnp.float32)
        lanes = lax.iota(jnp.int32, 16)

        def zdrow(i, _):
            for j in range(8):
                dloc[i, pl.ds(j * 16, 16)] = zv
            return 0
        lax.fori_loop(0, DROW, zdrow, 0)

        pltpu.sync_copy(z_hbm, acc_sp.at[pl.ds(sid * ZPT, ZPT)])
        plsc.subcore_barrier()

        pltpu.sync_copy(idx_hbm.at[wid, 0], idx_v.at[0])
        pltpu.async_copy(x_hbm.at[idx_v.at[0, 0]], xbuf.at[0], sem)

        def chunk(c, _):
            b = c & 1
            pltpu.make_async_copy(x_hbm.at[idx_v.at[b, 0]],
                                  xbuf.at[b], sem).wait()
            @pl.when(c + 1 < NCHK)
            def _():
                pltpu.sync_copy(idx_hbm.at[wid, c + 1], idx_v.at[1 - b])
                pltpu.async_copy(x_hbm.at[idx_v.at[1 - b, 0]],
                                 xbuf.at[1 - b], sem)
            pltpu.async_copy(rel_hbm.at[idx_v.at[b, 2]], relbuf,
                             sem2).wait()

            def group(g, _):
                dstvec = idx_v[b, 1, pl.ds(g * 16, 16)]
                for q in range(16):
                    k = g * 16 + q
                    for j in range(D // 16):
                        sl = pl.ds(j * 16, 16)
                        xbuf[b, k, sl] = xbuf[b, k, sl] * relbuf[k, sl]
                    d = dstvec[q]
                    r = d >> 7
                    c16 = ((d >> 4) & 7) << 4
                    oh = jnp.where(lanes == (d & 15), 1.0, 0.0)
                    csl = pl.ds(c16, 16)
                    dloc[r, csl] = dloc[r, csl] + oh
                return 0
            lax.fori_loop(0, CH // 16, group, 0)
            pltpu.sync_copy(xbuf.at[b], acc_sp.at[idx_v.at[b, 1]],
                            add=True)
            return 0
        lax.fori_loop(0, NCHK, chunk, 0)
        plsc.subcore_barrier()
        pltpu.sync_copy(acc_sp.at[pl.ds(sid * ZPT, ZPT)],
                        agg_out.at[cid, pl.ds(sid * ZPT, ZPT)])
        pltpu.sync_copy(dloc, deg_out.at[wid])

    return sc_kernel


def _tc_layer(NP, D, BM, NW):
    grid = (NP // BM,)
    row_spec = pl.BlockSpec((BM, D), lambda i: (i, 0))
    deg_spec = pl.BlockSpec((NW, BM // 128, 128), lambda i: (0, i, 0))
    w_spec = pl.BlockSpec((D, D), lambda i: (0, 0))
    s_spec = pl.BlockSpec((8, 128), lambda i: (0, 0))

    def body(a0, a1, dg, inr, wr, ws, sr, ho):
        s4 = jnp.sum(dg[...], axis=0)
        s_pad = jnp.concatenate(
            [s4, jnp.zeros((D - BM // 128, 128), jnp.float32)], axis=0)
        rown = lax.broadcasted_iota(jnp.int32, (BM, 128), 0)
        coln = lax.broadcasted_iota(jnp.int32, (BM, 128), 1)
        gsel = (coln == (rown >> 7)).astype(jnp.float32)
        rep = jnp.dot(gsel, s_pad, preferred_element_type=jnp.float32)
        msk = (coln == (rown & 127)).astype(jnp.float32)
        deg = jnp.dot(rep * msk, jnp.ones((128, 128), jnp.float32),
                      preferred_element_type=jnp.float32)
        deg = jnp.maximum(deg, 1.0)
        agg = (a0[...] + a1[...]) / deg
        h = (jnp.dot(agg, wr[...], preferred_element_type=jnp.float32) +
             jnp.dot(inr[...], ws[...], preferred_element_type=jnp.float32))
        ho[...] = jnp.maximum(h, 0.0) + sr[0:1, 0:1] * inr[...]

    return pl.pallas_call(
        body,
        grid=grid,
        in_specs=[row_spec, row_spec, deg_spec, row_spec,
                  w_spec, w_spec, s_spec],
        out_specs=row_spec,
        out_shape=jax.ShapeDtypeStruct((NP, D), jnp.float32),
    )


def kernel(x, edge_index, edge_type, rel_embeds, W_rel1, W_self1,
           W_rel2, W_self2):
    N, D = x.shape
    R = rel_embeds.shape[0]
    E = edge_index.shape[1]
    NW = NC * NS
    EPW = E // NW
    EPWP = -(-EPW // CH) * CH
    PAD = EPWP - EPW
    NCHK = EPWP // CH
    NP = -(-N // 256) * 256

    src = edge_index[0].astype(jnp.int32).reshape(NW, EPW)
    dst = edge_index[1].astype(jnp.int32).reshape(NW, EPW)
    et = edge_type.astype(jnp.int32).reshape(NW, EPW)
    if PAD:
        ar = jnp.arange(PAD, dtype=jnp.int32)
        pad_src = jnp.broadcast_to((ar * 97) % N, (NW, PAD))
        nbin = max(NP - N, 1)
        pad_dst = jnp.broadcast_to(min(N, NP - nbin) + (ar % nbin),
                                   (NW, PAD))
        pad_et = jnp.zeros((NW, PAD), jnp.int32)
        src = jnp.concatenate([src, pad_src], axis=1)
        dst = jnp.concatenate([dst, pad_dst], axis=1)
        et = jnp.concatenate([et, pad_et], axis=1)
    idx = jnp.stack([src.reshape(NW, NCHK, CH),
                     dst.reshape(NW, NCHK, CH),
                     et.reshape(NW, NCHK, CH)], axis=2)

    x_p = jnp.pad(x, ((0, NP - N), (0, 0)))
    zeros_init = jnp.zeros((NP // NS, D), jnp.float32)

    sc_agg = _build_sc_aggregate(NP, D, R, NCHK)
    tc = _tc_layer(NP, D, 1024, NW)

    w_rel = jnp.stack([W_rel1, W_rel2])
    w_self = jnp.stack([W_self1, W_self2])
    skip = jnp.stack([jnp.full((8, 128), 1.0, jnp.float32),
                      jnp.full((8, 128), 0.0, jnp.float32)])

    n_layers = jnp.minimum(edge_type[0].astype(jnp.int32), 0) + 2

    def layer(i, carry):
        cur, hsum = carry
        wr = lax.dynamic_index_in_dim(w_rel, i, keepdims=False)
        ws = lax.dynamic_index_in_dim(w_self, i, keepdims=False)
        sk = lax.dynamic_index_in_dim(skip, i, keepdims=False)
        aggp, degp = sc_agg(cur, idx, rel_embeds, zeros_init)
        h = tc(aggp[0], aggp[1], degp, cur, wr, ws, sk)
        return h, hsum + h

    _, hsum = lax.fori_loop(0, n_layers, layer, (x_p, jnp.zeros_like(x_p)))
    return hsum[:N] * 0.5


# speedup vs baseline: 1.4890x; 1.4890x over previous
import functools

import jax
import jax.numpy as jnp
from jax import lax
from jax.experimental import pallas as pl
from jax.experimental.pallas import tpu as pltpu
from jax.experimental.pallas import tpu_sc as plsc

NC = 2
NS = 16
CH = 80


def _build_sc_aggregate(NP, D, R, NCHK):
    ZPT = NP // NS
    DROW = NP // 128

    mesh = plsc.VectorSubcoreMesh(core_axis_name="c", subcore_axis_name="s")

    out_type = [
        jax.ShapeDtypeStruct((NC, NP, D), jnp.float32),
        jax.ShapeDtypeStruct((NC * NS, DROW, 128), jnp.float32),
    ]
    scratch = [
        pltpu.VMEM((3, CH), jnp.int32),
        pltpu.VMEM((3, CH), jnp.int32),
        pltpu.VMEM((CH, D), jnp.float32),
        pltpu.VMEM((CH, D), jnp.float32),
        pltpu.VMEM((CH, D), jnp.float32),
        pltpu.VMEM((DROW, 128), jnp.float32),
        pltpu.VMEM_SHARED((NP, D), jnp.float32),
        pltpu.SemaphoreType.DMA,
        pltpu.SemaphoreType.DMA,
        pltpu.SemaphoreType.DMA,
    ]

    @functools.partial(pl.kernel, out_type=out_type, mesh=mesh,
                       scratch_types=scratch)
    def sc_kernel(x_hbm, idx_hbm, rel_hbm, z_hbm, agg_out, deg_out,
                  idx_a, idx_b, relbuf, xbuf_a, xbuf_b, dloc, acc_sp,
                  sem_a, sem_b, sem2):
        cid = lax.axis_index("c")
        sid = lax.axis_index("s")
        wid = cid * NS + sid

        zv = jnp.zeros((16,), jnp.float32)
        lanes = lax.iota(jnp.int32, 16)

        def zdrow(i, _):
            for j in range(8):
                dloc[i, pl.ds(j * 16, 16)] = zv
            return 0
        lax.fori_loop(0, DROW, zdrow, 0)

        pltpu.sync_copy(z_hbm, acc_sp.at[pl.ds(sid * ZPT, ZPT)])
        plsc.subcore_barrier()

        def do_chunk(idx_v, xbuf, my_sem, other_idx, other_xbuf,
                     other_sem, c):
            pltpu.make_async_copy(x_hbm.at[idx_v.at[0]], xbuf,
                                  my_sem).wait()

            @pl.when(c + 1 < NCHK)
            def _():
                pltpu.sync_copy(idx_hbm.at[wid, c + 1], other_idx)
                pltpu.async_copy(x_hbm.at[other_idx.at[0]], other_xbuf,
                                 other_sem)
            pltpu.async_copy(rel_hbm.at[idx_v.at[2]], relbuf,
                             sem2).wait()

            def group(g, _):
                dstvec = idx_v[1, pl.ds(g * 16, 16)]
                for q in range(16):
                    k = g * 16 + q
                    for j in range(D // 16):
                        sl = pl.ds(j * 16, 16)
                        xbuf[k, sl] = xbuf[k, sl] * relbuf[k, sl]
                    d = dstvec[q]
                    r = d >> 7
                    c16 = ((d >> 4) & 7) << 4
                    oh = jnp.where(lanes == (d & 15), 1.0, 0.0)
                    csl = pl.ds(c16, 16)
                    dloc[r, csl] = dloc[r, csl] + oh
                return 0
            lax.fori_loop(0, CH // 16, group, 0)
            pltpu.sync_copy(xbuf, acc_sp.at[idx_v.at[1]], add=True)

        pltpu.sync_copy(idx_hbm.at[wid, 0], idx_a)
        pltpu.async_copy(x_hbm.at[idx_a.at[0]], xbuf_a, sem_a)

        def pair(p, _):
            c0 = 2 * p
            do_chunk(idx_a, xbuf_a, sem_a, idx_b, xbuf_b, sem_b, c0)

            @pl.when(c0 + 1 < NCHK)
            def _():
                do_chunk(idx_b, xbuf_b, sem_b, idx_a, xbuf_a, sem_a,
                         c0 + 1)
            return 0
        lax.fori_loop(0, (NCHK + 1) // 2, pair, 0)
        plsc.subcore_barrier()
        pltpu.sync_copy(acc_sp.at[pl.ds(sid * ZPT, ZPT)],
                        agg_out.at[cid, pl.ds(sid * ZPT, ZPT)])
        pltpu.sync_copy(dloc, deg_out.at[wid])

    return sc_kernel


def _tc_layer(NP, D, BM, NW):
    grid = (NP // BM,)
    row_spec = pl.BlockSpec((BM, D), lambda i: (i, 0))
    deg_spec = pl.BlockSpec((NW, BM // 128, 128), lambda i: (0, i, 0))
    w_spec = pl.BlockSpec((D, D), lambda i: (0, 0))
    s_spec = pl.BlockSpec((8, 128), lambda i: (0, 0))

    def body(a0, a1, dg, inr, wr, ws, sr, ho):
        s4 = jnp.sum(dg[...], axis=0)
        s_pad = jnp.concatenate(
            [s4, jnp.zeros((D - BM // 128, 128), jnp.float32)], axis=0)
        rown = lax.broadcasted_iota(jnp.int32, (BM, 128), 0)
        coln = lax.broadcasted_iota(jnp.int32, (BM, 128), 1)
        gsel = (coln == (rown >> 7)).astype(jnp.float32)
        rep = jnp.dot(gsel, s_pad, preferred_element_type=jnp.float32)
        msk = (coln == (rown & 127)).astype(jnp.float32)
        deg = jnp.dot(rep * msk, jnp.ones((128, 128), jnp.float32),
                      preferred_element_type=jnp.float32)
        deg = jnp.maximum(deg, 1.0)
        agg = (a0[...] + a1[...]) / deg
        h = (jnp.dot(agg, wr[...], preferred_element_type=jnp.float32) +
             jnp.dot(inr[...], ws[...], preferred_element_type=jnp.float32))
        ho[...] = jnp.maximum(h, 0.0) + sr[0:1, 0:1] * inr[...]

    return pl.pallas_call(
        body,
        grid=grid,
        in_specs=[row_spec, row_spec, deg_spec, row_spec,
                  w_spec, w_spec, s_spec],
        out_specs=row_spec,
        out_shape=jax.ShapeDtypeStruct((NP, D), jnp.float32),
    )


def kernel(x, edge_index, edge_type, rel_embeds, W_rel1, W_self1,
           W_rel2, W_self2):
    N, D = x.shape
    R = rel_embeds.shape[0]
    E = edge_index.shape[1]
    NW = NC * NS
    EPW = E // NW
    EPWP = -(-EPW // CH) * CH
    PAD = EPWP - EPW
    NCHK = EPWP // CH
    NP = -(-N // 256) * 256

    src = edge_index[0].astype(jnp.int32).reshape(NW, EPW)
    dst = edge_index[1].astype(jnp.int32).reshape(NW, EPW)
    et = edge_type.astype(jnp.int32).reshape(NW, EPW)
    if PAD:
        ar = jnp.arange(PAD, dtype=jnp.int32)
        pad_src = jnp.broadcast_to((ar * 97) % N, (NW, PAD))
        nbin = max(NP - N, 1)
        pad_dst = jnp.broadcast_to(min(N, NP - nbin) + (ar % nbin),
                                   (NW, PAD))
        pad_et = jnp.zeros((NW, PAD), jnp.int32)
        src = jnp.concatenate([src, pad_src], axis=1)
        dst = jnp.concatenate([dst, pad_dst], axis=1)
        et = jnp.concatenate([et, pad_et], axis=1)
    idx = jnp.stack([src.reshape(NW, NCHK, CH),
                     dst.reshape(NW, NCHK, CH),
                     et.reshape(NW, NCHK, CH)], axis=2)

    x_p = jnp.pad(x, ((0, NP - N), (0, 0)))
    zeros_init = jnp.zeros((NP // NS, D), jnp.float32)

    sc_agg = _build_sc_aggregate(NP, D, R, NCHK)
    tc = _tc_layer(NP, D, 1024, NW)

    w_rel = jnp.stack([W_rel1, W_rel2])
    w_self = jnp.stack([W_self1, W_self2])
    skip = jnp.stack([jnp.full((8, 128), 1.0, jnp.float32),
                      jnp.full((8, 128), 0.0, jnp.float32)])

    n_layers = jnp.minimum(edge_type[0].astype(jnp.int32), 0) + 2

    def layer(i, carry):
        cur, hsum = carry
        wr = lax.dynamic_index_in_dim(w_rel, i, keepdims=False)
        ws = lax.dynamic_index_in_dim(w_self, i, keepdims=False)
        sk = lax.dynamic_index_in_dim(skip, i, keepdims=False)
        aggp, degp = sc_agg(cur, idx, rel_embeds, zeros_init)
        h = tc(aggp[0], aggp[1], degp, cur, wr, ws, sk)
        return h, hsum + h

    _, hsum = lax.fori_loop(0, n_layers, layer, (x_p, jnp.zeros_like(x_p)))
    return hsum[:N] * 0.5


# speedup vs baseline: 1.5818x; 1.0624x over previous
import functools

import jax
import jax.numpy as jnp
from jax import lax
from jax.experimental import pallas as pl
from jax.experimental.pallas import tpu as pltpu
from jax.experimental.pallas import tpu_sc as plsc

NC = 2
NS = 16
CH = 80


def _build_sc_aggregate(NP, D, R, NCHK):
    ZPT = NP // NS
    DROW = NP // 128

    mesh = plsc.VectorSubcoreMesh(core_axis_name="c", subcore_axis_name="s")

    out_type = [
        jax.ShapeDtypeStruct((NC, NP, D), jnp.float32),
        jax.ShapeDtypeStruct((NC * NS, DROW, 128), jnp.float32),
    ]
    scratch = [
        pltpu.VMEM((3, CH), jnp.int32),
        pltpu.VMEM((CH, D), jnp.float32),
        pltpu.VMEM((CH, D), jnp.float32),
        pltpu.VMEM((DROW, 128), jnp.float32),
        pltpu.VMEM_SHARED((NP, D), jnp.float32),
        pltpu.VMEM_SHARED((R, D), jnp.float32),
        pltpu.SemaphoreType.DMA,
        pltpu.SemaphoreType.DMA,
    ]

    @functools.partial(pl.kernel, out_type=out_type, mesh=mesh,
                       scratch_types=scratch)
    def sc_kernel(x_hbm, idx_hbm, rel_hbm, z_hbm, agg_out, deg_out,
                  idx_v, relbuf, xbuf, dloc, acc_sp, rel_sp, sem, sem2):
        cid = lax.axis_index("c")
        sid = lax.axis_index("s")
        wid = cid * NS + sid

        zv = jnp.zeros((16,), jnp.float32)
        lanes = lax.iota(jnp.int32, 16)

        def zdrow(i, _):
            for j in range(8):
                dloc[i, pl.ds(j * 16, 16)] = zv
            return 0
        lax.fori_loop(0, DROW, zdrow, 0)

        pltpu.sync_copy(z_hbm, acc_sp.at[pl.ds(sid * ZPT, ZPT)])

        @pl.when(sid == 0)
        def _():
            pltpu.sync_copy(rel_hbm, rel_sp)
        plsc.subcore_barrier()

        def chunk(c, _):
            pltpu.sync_copy(idx_hbm.at[wid, c], idx_v)
            cpx = pltpu.async_copy(x_hbm.at[idx_v.at[0]], xbuf, sem)
            cpr = pltpu.async_copy(rel_sp.at[idx_v.at[2]], relbuf, sem2)
            cpx.wait()
            cpr.wait()

            def group(g, _):
                dstvec = idx_v[1, pl.ds(g * 16, 16)]
                for q in range(16):
                    k = g * 16 + q
                    for j in range(D // 16):
                        sl = pl.ds(j * 16, 16)
                        xbuf[k, sl] = xbuf[k, sl] * relbuf[k, sl]
                    d = dstvec[q]
                    r = d >> 7
                    c16 = ((d >> 4) & 7) << 4
                    oh = jnp.where(lanes == (d & 15), 1.0, 0.0)
                    csl = pl.ds(c16, 16)
                    dloc[r, csl] = dloc[r, csl] + oh
                return 0
            lax.fori_loop(0, CH // 16, group, 0)
            pltpu.sync_copy(xbuf, acc_sp.at[idx_v.at[1]], add=True)
            return 0
        lax.fori_loop(0, NCHK, chunk, 0)
        plsc.subcore_barrier()
        pltpu.sync_copy(acc_sp.at[pl.ds(sid * ZPT, ZPT)],
                        agg_out.at[cid, pl.ds(sid * ZPT, ZPT)])
        pltpu.sync_copy(dloc, deg_out.at[wid])

    return sc_kernel


def _tc_layer(NP, D, BM, NW):
    grid = (NP // BM,)
    row_spec = pl.BlockSpec((BM, D), lambda i: (i, 0))
    deg_spec = pl.BlockSpec((NW, BM // 128, 128), lambda i: (0, i, 0))
    w_spec = pl.BlockSpec((D, D), lambda i: (0, 0))
    s_spec = pl.BlockSpec((8, 128), lambda i: (0, 0))

    def body(a0, a1, dg, inr, wr, ws, sr, ho):
        s4 = jnp.sum(dg[...], axis=0)
        s_pad = jnp.concatenate(
            [s4, jnp.zeros((D - BM // 128, 128), jnp.float32)], axis=0)
        rown = lax.broadcasted_iota(jnp.int32, (BM, 128), 0)
        coln = lax.broadcasted_iota(jnp.int32, (BM, 128), 1)
        gsel = (coln == (rown >> 7)).astype(jnp.float32)
        rep = jnp.dot(gsel, s_pad, preferred_element_type=jnp.float32)
        msk = (coln == (rown & 127)).astype(jnp.float32)
        deg = jnp.dot(rep * msk, jnp.ones((128, 128), jnp.float32),
                      preferred_element_type=jnp.float32)
        deg = jnp.maximum(deg, 1.0)
        agg = (a0[...] + a1[...]) / deg
        h = (jnp.dot(agg, wr[...], preferred_element_type=jnp.float32) +
             jnp.dot(inr[...], ws[...], preferred_element_type=jnp.float32))
        ho[...] = jnp.maximum(h, 0.0) + sr[0:1, 0:1] * inr[...]

    return pl.pallas_call(
        body,
        grid=grid,
        in_specs=[row_spec, row_spec, deg_spec, row_spec,
                  w_spec, w_spec, s_spec],
        out_specs=row_spec,
        out_shape=jax.ShapeDtypeStruct((NP, D), jnp.float32),
    )


def kernel(x, edge_index, edge_type, rel_embeds, W_rel1, W_self1,
           W_rel2, W_self2):
    N, D = x.shape
    R = rel_embeds.shape[0]
    E = edge_index.shape[1]
    NW = NC * NS
    EPW = E // NW
    EPWP = -(-EPW // CH) * CH
    PAD = EPWP - EPW
    NCHK = EPWP // CH
    NP = -(-N // 256) * 256

    src = edge_index[0].astype(jnp.int32).reshape(NW, EPW)
    dst = edge_index[1].astype(jnp.int32).reshape(NW, EPW)
    et = edge_type.astype(jnp.int32).reshape(NW, EPW)
    if PAD:
        ar = jnp.arange(PAD, dtype=jnp.int32)
        pad_src = jnp.broadcast_to((ar * 97) % N, (NW, PAD))
        nbin = max(NP - N, 1)
        pad_dst = jnp.broadcast_to(min(N, NP - nbin) + (ar % nbin),
                                   (NW, PAD))
        pad_et = jnp.zeros((NW, PAD), jnp.int32)
        src = jnp.concatenate([src, pad_src], axis=1)
        dst = jnp.concatenate([dst, pad_dst], axis=1)
        et = jnp.concatenate([et, pad_et], axis=1)
    idx = jnp.stack([src.reshape(NW, NCHK, CH),
                     dst.reshape(NW, NCHK, CH),
                     et.reshape(NW, NCHK, CH)], axis=2)

    x_p = jnp.pad(x, ((0, NP - N), (0, 0)))
    zeros_init = jnp.zeros((NP // NS, D), jnp.float32)

    sc_agg = _build_sc_aggregate(NP, D, R, NCHK)
    tc = _tc_layer(NP, D, 1024, NW)

    w_rel = jnp.stack([W_rel1, W_rel2])
    w_self = jnp.stack([W_self1, W_self2])
    skip = jnp.stack([jnp.full((8, 128), 1.0, jnp.float32),
                      jnp.full((8, 128), 0.0, jnp.float32)])

    n_layers = jnp.minimum(edge_type[0].astype(jnp.int32), 0) + 2

    def layer(i, carry):
        cur, hsum = carry
        wr = lax.dynamic_index_in_dim(w_rel, i, keepdims=False)
        ws = lax.dynamic_index_in_dim(w_self, i, keepdims=False)
        sk = lax.dynamic_index_in_dim(skip, i, keepdims=False)
        aggp, degp = sc_agg(cur, idx, rel_embeds, zeros_init)
        h = tc(aggp[0], aggp[1], degp, cur, wr, ws, sk)
        return h, hsum + h

    _, hsum = lax.fori_loop(0, n_layers, layer, (x_p, jnp.zeros_like(x_p)))
    return hsum[:N] * 0.5


# speedup vs baseline: 1.7976x; 1.1364x over previous
import functools

import jax
import jax.numpy as jnp
from jax import lax
from jax.experimental import pallas as pl
from jax.experimental.pallas import tpu as pltpu
from jax.experimental.pallas import tpu_sc as plsc

NC = 2
NS = 16
CH = 80


def _build_sc_aggregate(NP, D, R, NCHK):
    ZPT = NP // NS
    DROW = NP // 128

    mesh = plsc.VectorSubcoreMesh(core_axis_name="c", subcore_axis_name="s")

    out_type = [
        jax.ShapeDtypeStruct((NC, NP, D), jnp.float32),
        jax.ShapeDtypeStruct((NC * NS, DROW, 128), jnp.float32),
    ]
    scratch = [
        pltpu.VMEM((3, CH), jnp.int32),
        pltpu.VMEM((3, CH), jnp.int32),
        pltpu.VMEM((CH, D), jnp.float32),
        pltpu.VMEM((CH, D), jnp.float32),
        pltpu.VMEM((CH, D), jnp.float32),
        pltpu.VMEM((DROW, 128), jnp.float32),
        pltpu.VMEM_SHARED((NP, D), jnp.float32),
        pltpu.VMEM_SHARED((R, D), jnp.float32),
        pltpu.SemaphoreType.DMA,
        pltpu.SemaphoreType.DMA,
        pltpu.SemaphoreType.DMA,
    ]

    @functools.partial(pl.kernel, out_type=out_type, mesh=mesh,
                       scratch_types=scratch)
    def sc_kernel(x_hbm, idx_hbm, rel_hbm, z_hbm, agg_out, deg_out,
                  idx_a, idx_b, relbuf, xbuf_a, xbuf_b, dloc, acc_sp,
                  rel_sp, sem_a, sem_b, sem2):
        cid = lax.axis_index("c")
        sid = lax.axis_index("s")
        wid = cid * NS + sid

        zv = jnp.zeros((16,), jnp.float32)
        lanes = lax.iota(jnp.int32, 16)

        def zdrow(i, _):
            for j in range(8):
                dloc[i, pl.ds(j * 16, 16)] = zv
            return 0
        lax.fori_loop(0, DROW, zdrow, 0)

        pltpu.sync_copy(z_hbm, acc_sp.at[pl.ds(sid * ZPT, ZPT)])

        @pl.when(sid == 0)
        def _():
            pltpu.sync_copy(rel_hbm, rel_sp)
        plsc.subcore_barrier()

        def do_chunk(idx_v, xbuf, my_sem, other_idx, other_xbuf,
                     other_sem, c):
            pltpu.make_async_copy(x_hbm.at[idx_v.at[0]], xbuf,
                                  my_sem).wait()

            @pl.when(c + 1 < NCHK)
            def _():
                pltpu.sync_copy(idx_hbm.at[wid, c + 1], other_idx)
                pltpu.async_copy(x_hbm.at[other_idx.at[0]], other_xbuf,
                                 other_sem)
            pltpu.async_copy(rel_sp.at[idx_v.at[2]], relbuf,
                             sem2).wait()

            def group(g, _):
                dstvec = idx_v[1, pl.ds(g * 16, 16)]
                for q in range(16):
                    k = g * 16 + q
                    for j in range(D // 16):
                        sl = pl.ds(j * 16, 16)
                        xbuf[k, sl] = xbuf[k, sl] * relbuf[k, sl]
                    d = dstvec[q]
                    r = d >> 7
                    c16 = ((d >> 4) & 7) << 4
                    oh = jnp.where(lanes == (d & 15), 1.0, 0.0)
                    csl = pl.ds(c16, 16)
                    dloc[r, csl] = dloc[r, csl] + oh
                return 0
            lax.fori_loop(0, CH // 16, group, 0)
            pltpu.sync_copy(xbuf, acc_sp.at[idx_v.at[1]], add=True)

        pltpu.sync_copy(idx_hbm.at[wid, 0], idx_a)
        pltpu.async_copy(x_hbm.at[idx_a.at[0]], xbuf_a, sem_a)

        def pair(p, _):
            c0 = 2 * p
            do_chunk(idx_a, xbuf_a, sem_a, idx_b, xbuf_b, sem_b, c0)

            @pl.when(c0 + 1 < NCHK)
            def _():
                do_chunk(idx_b, xbuf_b, sem_b, idx_a, xbuf_a, sem_a,
                         c0 + 1)
            return 0
        lax.fori_loop(0, (NCHK + 1) // 2, pair, 0)
        plsc.subcore_barrier()
        pltpu.sync_copy(acc_sp.at[pl.ds(sid * ZPT, ZPT)],
                        agg_out.at[cid, pl.ds(sid * ZPT, ZPT)])
        pltpu.sync_copy(dloc, deg_out.at[wid])

    return sc_kernel


def _tc_layer(NP, D, BM, NW):
    grid = (NP // BM,)
    row_spec = pl.BlockSpec((BM, D), lambda i: (i, 0))
    deg_spec = pl.BlockSpec((NW, BM // 128, 128), lambda i: (0, i, 0))
    w_spec = pl.BlockSpec((D, D), lambda i: (0, 0))
    s_spec = pl.BlockSpec((8, 128), lambda i: (0, 0))

    def body(a0, a1, dg, inr, wr, ws, sr, ho):
        s4 = jnp.sum(dg[...], axis=0)
        s_pad = jnp.concatenate(
            [s4, jnp.zeros((D - BM // 128, 128), jnp.float32)], axis=0)
        rown = lax.broadcasted_iota(jnp.int32, (BM, 128), 0)
        coln = lax.broadcasted_iota(jnp.int32, (BM, 128), 1)
        gsel = (coln == (rown >> 7)).astype(jnp.float32)
        rep = jnp.dot(gsel, s_pad, preferred_element_type=jnp.float32)
        msk = (coln == (rown & 127)).astype(jnp.float32)
        deg = jnp.dot(rep * msk, jnp.ones((128, 128), jnp.float32),
                      preferred_element_type=jnp.float32)
        deg = jnp.maximum(deg, 1.0)
        agg = (a0[...] + a1[...]) / deg
        h = (jnp.dot(agg, wr[...], preferred_element_type=jnp.float32) +
             jnp.dot(inr[...], ws[...], preferred_element_type=jnp.float32))
        ho[...] = jnp.maximum(h, 0.0) + sr[0:1, 0:1] * inr[...]

    return pl.pallas_call(
        body,
        grid=grid,
        in_specs=[row_spec, row_spec, deg_spec, row_spec,
                  w_spec, w_spec, s_spec],
        out_specs=row_spec,
        out_shape=jax.ShapeDtypeStruct((NP, D), jnp.float32),
    )


def kernel(x, edge_index, edge_type, rel_embeds, W_rel1, W_self1,
           W_rel2, W_self2):
    N, D = x.shape
    R = rel_embeds.shape[0]
    E = edge_index.shape[1]
    NW = NC * NS
    EPW = E // NW
    EPWP = -(-EPW // CH) * CH
    PAD = EPWP - EPW
    NCHK = EPWP // CH
    NP = -(-N // 256) * 256

    src = edge_index[0].astype(jnp.int32).reshape(NW, EPW)
    dst = edge_index[1].astype(jnp.int32).reshape(NW, EPW)
    et = edge_type.astype(jnp.int32).reshape(NW, EPW)
    if PAD:
        ar = jnp.arange(PAD, dtype=jnp.int32)
        pad_src = jnp.broadcast_to((ar * 97) % N, (NW, PAD))
        nbin = max(NP - N, 1)
        pad_dst = jnp.broadcast_to(min(N, NP - nbin) + (ar % nbin),
                                   (NW, PAD))
        pad_et = jnp.zeros((NW, PAD), jnp.int32)
        src = jnp.concatenate([src, pad_src], axis=1)
        dst = jnp.concatenate([dst, pad_dst], axis=1)
        et = jnp.concatenate([et, pad_et], axis=1)
    idx = jnp.stack([src.reshape(NW, NCHK, CH),
                     dst.reshape(NW, NCHK, CH),
                     et.reshape(NW, NCHK, CH)], axis=2)

    x_p = jnp.pad(x, ((0, NP - N), (0, 0)))
    zeros_init = jnp.zeros((NP // NS, D), jnp.float32)

    sc_agg = _build_sc_aggregate(NP, D, R, NCHK)
    tc = _tc_layer(NP, D, 1024, NW)

    w_rel = jnp.stack([W_rel1, W_rel2])
    w_self = jnp.stack([W_self1, W_self2])
    skip = jnp.stack([jnp.full((8, 128), 1.0, jnp.float32),
                      jnp.full((8, 128), 0.0, jnp.float32)])

    n_layers = jnp.minimum(edge_type[0].astype(jnp.int32), 0) + 2

    def layer(i, carry):
        cur, hsum = carry
        wr = lax.dynamic_index_in_dim(w_rel, i, keepdims=False)
        ws = lax.dynamic_index_in_dim(w_self, i, keepdims=False)
        sk = lax.dynamic_index_in_dim(skip, i, keepdims=False)
        aggp, degp = sc_agg(cur, idx, rel_embeds, zeros_init)
        h = tc(aggp[0], aggp[1], degp, cur, wr, ws, sk)
        return h, hsum + h

    _, hsum = lax.fori_loop(0, n_layers, layer, (x_p, jnp.zeros_like(x_p)))
    return hsum[:N] * 0.5


# speedup vs baseline: 1.7988x; 1.0007x over previous
import functools

import jax
import jax.numpy as jnp
from jax import lax
from jax.experimental import pallas as pl
from jax.experimental.pallas import tpu as pltpu
from jax.experimental.pallas import tpu_sc as plsc

NC = 2
NS = 16
CH = 80


def _build_sc_aggregate(NP, D, R, NCHK):
    ZPT = NP // NS
    DROW = NP // 128

    mesh = plsc.VectorSubcoreMesh(core_axis_name="c", subcore_axis_name="s")

    out_type = [
        jax.ShapeDtypeStruct((NC, NP, D), jnp.float32),
        jax.ShapeDtypeStruct((NC * NS, DROW, 128), jnp.float32),
    ]
    scratch = [
        pltpu.VMEM((3, CH), jnp.int32),
        pltpu.VMEM((3, CH), jnp.int32),
        pltpu.VMEM((CH, D), jnp.float32),
        pltpu.VMEM((CH, D), jnp.float32),
        pltpu.VMEM((CH, D), jnp.float32),
        pltpu.VMEM((DROW, 128), jnp.float32),
        pltpu.VMEM_SHARED((NP, D), jnp.float32),
        pltpu.VMEM_SHARED((R, D), jnp.float32),
        pltpu.SemaphoreType.DMA,
        pltpu.SemaphoreType.DMA,
        pltpu.SemaphoreType.DMA,
        pltpu.SemaphoreType.DMA,
        pltpu.SemaphoreType.DMA,
    ]

    @functools.partial(pl.kernel, out_type=out_type, mesh=mesh,
                       scratch_types=scratch)
    def sc_kernel(x_hbm, idx_hbm, rel_hbm, z_hbm, agg_out, deg_out,
                  idx_a, idx_b, relbuf, xbuf_a, xbuf_b, dloc, acc_sp,
                  rel_sp, sem_a, sem_b, ssem_a, ssem_b, sem2):
        cid = lax.axis_index("c")
        sid = lax.axis_index("s")
        wid = cid * NS + sid

        zv = jnp.zeros((16,), jnp.float32)
        lanes = lax.iota(jnp.int32, 16)

        def zdrow(i, _):
            for j in range(8):
                dloc[i, pl.ds(j * 16, 16)] = zv
            return 0
        lax.fori_loop(0, DROW, zdrow, 0)

        pltpu.sync_copy(z_hbm, acc_sp.at[pl.ds(sid * ZPT, ZPT)])

        @pl.when(sid == 0)
        def _():
            pltpu.sync_copy(rel_hbm, rel_sp)
        plsc.subcore_barrier()

        def do_chunk(idx_v, xbuf, my_sem, my_ssem, other_idx,
                     other_xbuf, other_sem, other_ssem, c):
            pltpu.make_async_copy(x_hbm.at[idx_v.at[0]], xbuf,
                                  my_sem).wait()

            @pl.when(c + 1 < NCHK)
            def _():
                @pl.when(c >= 1)
                def _():
                    pltpu.make_async_copy(
                        other_xbuf, acc_sp.at[pl.ds(0, CH)],
                        other_ssem).wait()
                pltpu.sync_copy(idx_hbm.at[wid, c + 1], other_idx)
                pltpu.async_copy(x_hbm.at[other_idx.at[0]], other_xbuf,
                                 other_sem)
            pltpu.async_copy(rel_sp.at[idx_v.at[2]], relbuf,
                             sem2).wait()

            def group(g, _):
                dstvec = idx_v[1, pl.ds(g * 16, 16)]
                for q in range(16):
                    k = g * 16 + q
                    for j in range(D // 16):
                        sl = pl.ds(j * 16, 16)
                        xbuf[k, sl] = xbuf[k, sl] * relbuf[k, sl]
                    d = dstvec[q]
                    r = d >> 7
                    c16 = ((d >> 4) & 7) << 4
                    oh = jnp.where(lanes == (d & 15), 1.0, 0.0)
                    csl = pl.ds(c16, 16)
                    dloc[r, csl] = dloc[r, csl] + oh
                return 0
            lax.fori_loop(0, CH // 16, group, 0)
            pltpu.async_copy(xbuf, acc_sp.at[idx_v.at[1]], my_ssem,
                             add=True)

        pltpu.sync_copy(idx_hbm.at[wid, 0], idx_a)
        pltpu.async_copy(x_hbm.at[idx_a.at[0]], xbuf_a, sem_a)

        def pair(p, _):
            c0 = 2 * p
            do_chunk(idx_a, xbuf_a, sem_a, ssem_a, idx_b, xbuf_b,
                     sem_b, ssem_b, c0)

            @pl.when(c0 + 1 < NCHK)
            def _():
                do_chunk(idx_b, xbuf_b, sem_b, ssem_b, idx_a, xbuf_a,
                         sem_a, ssem_a, c0 + 1)
            return 0
        lax.fori_loop(0, (NCHK + 1) // 2, pair, 0)
        pltpu.make_async_copy(xbuf_b, acc_sp.at[pl.ds(0, CH)],
                              ssem_b).wait()
        pltpu.make_async_copy(xbuf_a, acc_sp.at[pl.ds(0, CH)],
                              ssem_a).wait()
        plsc.subcore_barrier()
        pltpu.sync_copy(acc_sp.at[pl.ds(sid * ZPT, ZPT)],
                        agg_out.at[cid, pl.ds(sid * ZPT, ZPT)])
        pltpu.sync_copy(dloc, deg_out.at[wid])

    return sc_kernel


def _tc_layer(NP, D, BM, NW):
    grid = (NP // BM,)
    row_spec = pl.BlockSpec((BM, D), lambda i: (i, 0))
    deg_spec = pl.BlockSpec((NW, BM // 128, 128), lambda i: (0, i, 0))
    w_spec = pl.BlockSpec((D, D), lambda i: (0, 0))
    s_spec = pl.BlockSpec((8, 128), lambda i: (0, 0))

    def body(a0, a1, dg, inr, wr, ws, sr, ho):
        s4 = jnp.sum(dg[...], axis=0)
        s_pad = jnp.concatenate(
            [s4, jnp.zeros((D - BM // 128, 128), jnp.float32)], axis=0)
        rown = lax.broadcasted_iota(jnp.int32, (BM, 128), 0)
        coln = lax.broadcasted_iota(jnp.int32, (BM, 128), 1)
        gsel = (coln == (rown >> 7)).astype(jnp.float32)
        rep = jnp.dot(gsel, s_pad, preferred_element_type=jnp.float32)
        msk = (coln == (rown & 127)).astype(jnp.float32)
        deg = jnp.dot(rep * msk, jnp.ones((128, 128), jnp.float32),
                      preferred_element_type=jnp.float32)
        deg = jnp.maximum(deg, 1.0)
        agg = (a0[...] + a1[...]) / deg
        h = (jnp.dot(agg, wr[...], preferred_element_type=jnp.float32) +
             jnp.dot(inr[...], ws[...], preferred_element_type=jnp.float32))
        ho[...] = jnp.maximum(h, 0.0) + sr[0:1, 0:1] * inr[...]

    return pl.pallas_call(
        body,
        grid=grid,
        in_specs=[row_spec, row_spec, deg_spec, row_spec,
                  w_spec, w_spec, s_spec],
        out_specs=row_spec,
        out_shape=jax.ShapeDtypeStruct((NP, D), jnp.float32),
    )


def kernel(x, edge_index, edge_type, rel_embeds, W_rel1, W_self1,
           W_rel2, W_self2):
    N, D = x.shape
    R = rel_embeds.shape[0]
    E = edge_index.shape[1]
    NW = NC * NS
    EPW = E // NW
    EPWP = -(-EPW // CH) * CH
    PAD = EPWP - EPW
    NCHK = EPWP // CH
    NP = -(-N // 256) * 256

    src = edge_index[0].astype(jnp.int32).reshape(NW, EPW)
    dst = edge_index[1].astype(jnp.int32).reshape(NW, EPW)
    et = edge_type.astype(jnp.int32).reshape(NW, EPW)
    if PAD:
        ar = jnp.arange(PAD, dtype=jnp.int32)
        pad_src = jnp.broadcast_to((ar * 97) % N, (NW, PAD))
        nbin = max(NP - N, 1)
        pad_dst = jnp.broadcast_to(min(N, NP - nbin) + (ar % nbin),
                                   (NW, PAD))
        pad_et = jnp.zeros((NW, PAD), jnp.int32)
        src = jnp.concatenate([src, pad_src], axis=1)
        dst = jnp.concatenate([dst, pad_dst], axis=1)
        et = jnp.concatenate([et, pad_et], axis=1)
    idx = jnp.stack([src.reshape(NW, NCHK, CH),
                     dst.reshape(NW, NCHK, CH),
                     et.reshape(NW, NCHK, CH)], axis=2)

    x_p = jnp.pad(x, ((0, NP - N), (0, 0)))
    zeros_init = jnp.zeros((NP // NS, D), jnp.float32)

    sc_agg = _build_sc_aggregate(NP, D, R, NCHK)
    tc = _tc_layer(NP, D, 1024, NW)

    w_rel = jnp.stack([W_rel1, W_rel2])
    w_self = jnp.stack([W_self1, W_self2])
    skip = jnp.stack([jnp.full((8, 128), 1.0, jnp.float32),
                      jnp.full((8, 128), 0.0, jnp.float32)])

    n_layers = jnp.minimum(edge_type[0].astype(jnp.int32), 0) + 2

    def layer(i, carry):
        cur, hsum = carry
        wr = lax.dynamic_index_in_dim(w_rel, i, keepdims=False)
        ws = lax.dynamic_index_in_dim(w_self, i, keepdims=False)
        sk = lax.dynamic_index_in_dim(skip, i, keepdims=False)
        aggp, degp = sc_agg(cur, idx, rel_embeds, zeros_init)
        h = tc(aggp[0], aggp[1], degp, cur, wr, ws, sk)
        return h, hsum + h

    _, hsum = lax.fori_loop(0, n_layers, layer, (x_p, jnp.zeros_like(x_p)))
    return hsum[:N] * 0.5
